# baseline (speedup 1.0000x reference)
import functools

import jax
import jax.numpy as jnp
from jax import lax
from jax.experimental import pallas as pl
from jax.experimental.pallas import tpu as pltpu
from jax.experimental.pallas import tpu_sc as plsc

D = 32
NC = 2
NS = 16
NW = NC * NS
CHUNK = 128

V_ROWS = 100000
TILES = V_ROWS // 8
LINES = V_ROWS // 4
TPW = 392
CTILES = 28
CROWS = CTILES * 8
CLINES = CROWS // 4
CPAIRS = TPW // CTILES // 2


def _sc_pack6():
    mesh = plsc.VectorSubcoreMesh(core_axis_name="c", subcore_axis_name="s")
    out_type = [jax.ShapeDtypeStruct((LINES, 128), jnp.float32)] * 6
    scratch_types = [
        pltpu.VMEM((CROWS, D), jnp.float32),
        pltpu.VMEM((CROWS, D), jnp.float32),
        pltpu.VMEM((CLINES, 128), jnp.float32),
        pltpu.VMEM((CLINES, 128), jnp.float32),
        pltpu.SemaphoreType.DMA,
        pltpu.SemaphoreType.DMA,
        pltpu.SemaphoreType.DMA,
        pltpu.SemaphoreType.DMA,
    ]

    @functools.partial(pl.kernel, mesh=mesh, out_type=out_type,
                       scratch_types=scratch_types)
    def k(t0, t1, t2, t3, t4, t5, o0, o1, o2, o3, o4, o5,
          in0, in1, pk0, pk1, si0, si1, so0, so1):
        wid = lax.axis_index("s") * NC + lax.axis_index("c")
        tbls = (t0, t1, t2, t3, t4, t5)
        outs = (o0, o1, o2, o3, o4, o5)
        ins = (in0, in1)
        pks = (pk0, pk1)
        sis = (si0, si1)
        sos = (so0, so1)
        t_base = wid * TPW

        def t0_of(c):
            return jnp.minimum(t_base + c * CTILES, TILES - CTILES)

        def repack(bin_, bout):
            def body(i, carry):
                for l4 in range(4):
                    ln = 4 * i + l4
                    for rr in range(4):
                        r = 4 * ln + rr
                        bout[ln, pl.ds(32 * rr, 16)] = bin_[r, pl.ds(0, 16)]
                        bout[ln, pl.ds(32 * rr + 16, 16)] = (
                            bin_[r, pl.ds(16, 16)])
                return carry

            lax.fori_loop(0, CLINES // 4, body, 0)

        for t in range(6):
            tbl, out = tbls[t], outs[t]

            def pair(c2, carry, tbl=tbl, out=out):
                hs = []
                for b in range(2):
                    r0 = pl.multiple_of(t0_of(2 * c2 + b) * 8, 32)
                    hs.append(pltpu.async_copy(tbl.at[pl.ds(r0, CROWS)],
                                               ins[b], sis[b]))
                for b in range(2):
                    hs[b].wait()
                    repack(ins[b], pks[b])
                    line0 = pl.multiple_of(t0_of(2 * c2 + b) * 2, 8)
                    pltpu.sync_copy(pks[b], out.at[pl.ds(line0, CLINES)])
                return carry

            lax.fori_loop(0, CPAIRS, pair, 0)

    return k


def _sc_gather8(B):
    b_per_w = B // NW
    n_chunks = b_per_w // CHUNK
    NBUF = 2
    mesh = plsc.VectorSubcoreMesh(core_axis_name="c", subcore_axis_name="s")
    out_type = [jax.ShapeDtypeStruct((B, D), jnp.float32)] * 8
    scratch_types = (
        [pltpu.VMEM((n_chunks, CHUNK), jnp.int32) for _ in range(8)]
        + [pltpu.VMEM((CHUNK, 128), jnp.float32) for _ in range(NBUF)]
        + [pltpu.VMEM((CHUNK, D), jnp.float32)]
        + [pltpu.SemaphoreType.DMA for _ in range(NBUF)]
    )

    @functools.partial(pl.kernel, mesh=mesh, out_type=out_type,
                       scratch_types=scratch_types,
                       compiler_params=pltpu.CompilerParams(
                           use_tc_tiling_on_sc=False))
    def k(aeu, teu, aei, tei, wa, wb,
          lau_h, ltu_h, lai_h, lti_h, oau_h, otu_h, oai_h, oti_h,
          o0, o1, o2, o3, o4, o5, o6, o7, *scr):
        lns = scr[0:4]
        ofs = scr[4:8]
        bufs = scr[8:8 + NBUF]
        cb = scr[8 + NBUF]
        sems = scr[9 + NBUF:9 + 2 * NBUF]
        wid = lax.axis_index("s") * NC + lax.axis_index("c")
        base = wid * b_per_w
        for h, v in zip((lau_h, ltu_h, lai_h, lti_h,
                         oau_h, otu_h, oai_h, oti_h), lns + ofs):
            pltpu.sync_copy(h.at[wid], v)

        plan = [(aeu, 0), (teu, 1), (aei, 2), (tei, 3),
                (wa, 0), (wa, 1), (wb, 0), (wb, 1)]
        outs = [o0, o1, o2, o3, o4, o5, o6, o7]

        for g in range(8):
            tbl, iset = plan[g]
            out, lref, oref = outs[g], lns[iset], ofs[iset]

            def pair(c2, carry, tbl=tbl, out=out, lref=lref, oref=oref):
                hs = []
                for b in range(2):
                    hs.append(pltpu.async_copy(tbl.at[lref.at[2 * c2 + b]],
                                               bufs[b], sems[b]))
                for b in range(2):
                    c = 2 * c2 + b
                    hs[b].wait()
                    lbuf = bufs[b]

                    def body(i, carry2, lbuf=lbuf, c=c, oref=oref):
                        offv = oref[c, pl.ds(16 * i, 16)]
                        for rr in range(16):
                            r = 16 * i + rr
                            off = offv[rr]
                            cb[r, pl.ds(0, 16)] = lbuf[r, pl.ds(off, 16)]
                            cb[r, pl.ds(16, 16)] = (
                                lbuf[r, pl.ds(off + 16, 16)])
                        return carry2

                    lax.fori_loop(0, CHUNK // 16, body, 0)
                    pltpu.sync_copy(
                        cb, out.at[pl.ds(base + c * CHUNK, CHUNK)])
                return carry

            lax.fori_loop(0, n_chunks // 2, pair, 0)

    return k


def _tc_combine(nb, inv_b):
    def body(au, tu, ai, ti, waau, watu, wbau, wbtu, ar, tr,
             w1a, b1a, w2a, b2a, w1t, b1t, w2t, b2t,
             w1i, b1i, w2i, b2i, w1j, b1j, w2j, b2j, la, lt):
        x_au = waau[...] * au[...] + (1.0 - watu[...]) * tu[...]
        x_tu = wbau[...] * au[...] + (1.0 - wbtu[...]) * tu[...]

        def mlp(x, w1, b1, w2, b2):
            h = jnp.maximum(
                jnp.dot(x, w1[...], preferred_element_type=jnp.float32)
                + b1[...], 0.0)
            return jnp.maximum(
                jnp.dot(h, w2[...], preferred_element_type=jnp.float32)
                + b2[...], 0.0)

        f_au = mlp(x_au, w1a, b1a, w2a, b2a)
        f_tu = mlp(x_tu, w1t, b1t, w2t, b2t)
        f_ai = mlp(ai[...], w1i, b1i, w2i, b2i)
        f_ti = mlp(ti[...], w1j, b1j, w2j, b2j)

        a_dot = jnp.sum(f_au * f_ai, axis=1, keepdims=True)
        t_dot = jnp.sum(f_tu * f_ti, axis=1, keepdims=True)
        a_s = jnp.maximum(a_dot, jnp.float32(1e-06))
        t_s = jnp.maximum(t_dot, jnp.float32(1e-06))
        pa = jnp.sum((a_s - ar[...]) ** 2) * inv_b
        pt = jnp.sum((t_s - tr[...]) ** 2) * inv_b

        i = pl.program_id(0)

        @pl.when(i == 0)
        def _():
            la[0, 0] = jnp.float32(0.0)
            lt[0, 0] = jnp.float32(0.0)

        la[0, 0] += pa
        lt[0, 0] += pt

    return body


def kernel(ausers, aitems, aratings, tusers, titems, tratings, params):
    B = ausers.shape[0]
    assert B % (NW * CHUNK) == 0
    n_chunks = (B // NW) // CHUNK

    tables = (params["a_emb_user"], params["t_emb_user"],
              params["a_emb_item"], params["t_emb_item"],
              params["W_a"], params["W_b"])

    packed = _sc_pack6()(*tables)

    iau, itu, iai, iti = (a.astype(jnp.int32)
                          for a in (ausers, tusers, aitems, titems))
    lines = [(a // 4).reshape(NW, n_chunks, CHUNK)
             for a in (iau, itu, iai, iti)]
    offs = [((a % 4) * D).reshape(NW, n_chunks, CHUNK)
            for a in (iau, itu, iai, iti)]

    g = _sc_gather8(B)(*packed, *lines, *offs)
    a_u, t_u, a_i, t_i, wa_au, wa_tu, wb_au, wb_tu = g

    NB = 8
    R = B // NB
    row = pl.BlockSpec((R, D), lambda i: (i, 0))
    col = pl.BlockSpec((R, 1), lambda i: (i, 0))

    def full(shape):
        return pl.BlockSpec(shape, lambda i: tuple(0 for _ in shape))

    mlps = (params["mlp_a_users"], params["mlp_t_users"],
            params["mlp_a_items"], params["mlp_t_items"])
    wargs, wspecs = [], []
    for p in mlps:
        for arr in (p["W1"], p["b1"].reshape(1, -1),
                    p["W2"], p["b2"].reshape(1, -1)):
            wargs.append(arr)
            wspecs.append(full(arr.shape))

    ar2 = aratings.astype(jnp.float32).reshape(B, 1)
    tr2 = tratings.astype(jnp.float32).reshape(B, 1)

    la, lt = pl.pallas_call(
        _tc_combine(NB, 1.0 / B),
        grid=(NB,),
        in_specs=[row] * 8 + [col, col] + wspecs,
        out_specs=(pl.BlockSpec((1, 1), lambda i: (0, 0),
                                memory_space=pltpu.SMEM),) * 2,
        out_shape=(jax.ShapeDtypeStruct((1, 1), jnp.float32),) * 2,
    )(a_u, t_u, a_i, t_i, wa_au, wa_tu, wb_au, wb_tu, ar2, tr2, *wargs)

    return (la[0, 0], lt[0, 0])


# speedup vs baseline: 1.0377x; 1.0377x over previous
import functools

import jax
import jax.numpy as jnp
from jax import lax
from jax.experimental import pallas as pl
from jax.experimental.pallas import tpu as pltpu
from jax.experimental.pallas import tpu_sc as plsc

D = 32
NC = 2
NS = 16
NW = NC * NS
CHUNK = 128

V_ROWS = 100000
TILES = V_ROWS // 8
LINES = V_ROWS // 4
TPW = 392
CTILES = 28
CROWS = CTILES * 8
CLINES = CROWS // 4
CPAIRS = TPW // CTILES // 2


def _sc_pack6():
    mesh = plsc.VectorSubcoreMesh(core_axis_name="c", subcore_axis_name="s")
    out_type = [jax.ShapeDtypeStruct((LINES, 128), jnp.float32)] * 6
    scratch_types = [
        pltpu.VMEM((CROWS, D), jnp.float32),
        pltpu.VMEM((CROWS, D), jnp.float32),
        pltpu.VMEM((CLINES, 128), jnp.float32),
        pltpu.VMEM((CLINES, 128), jnp.float32),
        pltpu.SemaphoreType.DMA,
        pltpu.SemaphoreType.DMA,
    ]

    @functools.partial(pl.kernel, mesh=mesh, out_type=out_type,
                       scratch_types=scratch_types)
    def k(t0, t1, t2, t3, t4, t5, o0, o1, o2, o3, o4, o5,
          in0, in1, pk0, pk1, si0, si1):
        wid = lax.axis_index("s") * NC + lax.axis_index("c")
        tbls = (t0, t1, t2, t3, t4, t5)
        outs = (o0, o1, o2, o3, o4, o5)
        ins = (in0, in1)
        pks = (pk0, pk1)
        sis = (si0, si1)
        t_base = wid * TPW

        def t0_of(c):
            return jnp.minimum(t_base + c * CTILES, TILES - CTILES)

        def repack(bin_, bout):
            def body(i, carry):
                for rr in range(8):
                    r = 8 * i + rr
                    ln = 2 * i + rr // 4
                    c0 = 32 * (rr % 4)
                    bout[ln, pl.ds(c0, 16)] = bin_[r, pl.ds(0, 16)]
                    bout[ln, pl.ds(c0 + 16, 16)] = bin_[r, pl.ds(16, 16)]
                return carry

            lax.fori_loop(0, CTILES, body, 0)

        for t in range(6):
            tbl, out = tbls[t], outs[t]

            def pair(c2, carry, tbl=tbl, out=out):
                hs = []
                for b in range(2):
                    r0 = pl.multiple_of(t0_of(2 * c2 + b) * 8, 32)
                    hs.append(pltpu.async_copy(tbl.at[pl.ds(r0, CROWS)],
                                               ins[b], sis[b]))
                for b in range(2):
                    hs[b].wait()
                    repack(ins[b], pks[b])
                    line0 = pl.multiple_of(t0_of(2 * c2 + b) * 2, 8)
                    pltpu.sync_copy(pks[b], out.at[pl.ds(line0, CLINES)])
                return carry

            lax.fori_loop(0, CPAIRS, pair, 0)

    return k


def _sc_gather8(B):
    b_per_w = B // NW
    n_chunks = b_per_w // CHUNK
    mesh = plsc.VectorSubcoreMesh(core_axis_name="c", subcore_axis_name="s")
    out_type = [jax.ShapeDtypeStruct((B // 8, 8, D), jnp.float32)] * 8
    scratch_types = (
        [pltpu.VMEM((8, CHUNK), jnp.int32) for _ in range(8)]
        + [pltpu.VMEM((CHUNK, 128), jnp.float32) for _ in range(2)]
        + [pltpu.VMEM((CHUNK // 8, 8, D), jnp.float32)]
        + [pltpu.SemaphoreType.DMA for _ in range(2)]
    )

    @functools.partial(pl.kernel, mesh=mesh, out_type=out_type,
                       scratch_types=scratch_types)
    def k(aeu, teu, aei, tei, wa, wb,
          lau_h, ltu_h, lai_h, lti_h, oau_h, otu_h, oai_h, oti_h,
          o0, o1, o2, o3, o4, o5, o6, o7, *scr):
        lns = scr[0:4]
        ofs = scr[4:8]
        bufs = scr[8:10]
        cb = scr[10]
        sems = scr[11:13]
        wid = lax.axis_index("s") * NC + lax.axis_index("c")
        base = wid * b_per_w
        hwid = wid // 2
        row0 = (wid % 2) * 4
        for h, v in zip((lau_h, ltu_h, lai_h, lti_h,
                         oau_h, otu_h, oai_h, oti_h), lns + ofs):
            pltpu.sync_copy(h.at[hwid], v)

        plan = [(aeu, 0), (teu, 1), (aei, 2), (tei, 3),
                (wa, 0), (wa, 1), (wb, 0), (wb, 1)]
        outs = [o0, o1, o2, o3, o4, o5, o6, o7]

        for g in range(8):
            tbl, iset = plan[g]
            out, lref, oref = outs[g], lns[iset], ofs[iset]

            def pair(c2, carry, tbl=tbl, out=out, lref=lref, oref=oref):
                hs = []
                for b in range(2):
                    hs.append(pltpu.async_copy(
                        tbl.at[lref.at[row0 + 2 * c2 + b]],
                        bufs[b], sems[b]))
                for b in range(2):
                    c = 2 * c2 + b
                    hs[b].wait()
                    lbuf = bufs[b]

                    def body(j, carry2, lbuf=lbuf, c=c, oref=oref):
                        offv = oref[row0 + c, pl.ds(16 * j, 16)]
                        for rr in range(16):
                            r = 16 * j + rr
                            off = offv[rr]
                            ct, cr = 2 * j + rr // 8, rr % 8
                            cb[ct, cr, pl.ds(0, 16)] = (
                                lbuf[r, pl.ds(off, 16)])
                            cb[ct, cr, pl.ds(16, 16)] = (
                                lbuf[r, pl.ds(off + 16, 16)])
                        return carry2

                    lax.fori_loop(0, CHUNK // 16, body, 0)
                    t0 = (base + c * CHUNK) // 8
                    pltpu.sync_copy(cb, out.at[pl.ds(t0, CHUNK // 8)])
                return carry

            lax.fori_loop(0, n_chunks // 2, pair, 0)

    return k


def _tc_combine(nb, inv_b, R):
    def body(au, tu, ai, ti, waau, watu, wbau, wbtu, ar, tr,
             w1a, b1a, w2a, b2a, w1t, b1t, w2t, b2t,
             w1i, b1i, w2i, b2i, w1j, b1j, w2j, b2j, la, lt):
        def rd(ref):
            return ref[...].reshape(R, D)

        x_au = rd(waau) * rd(au) + (1.0 - rd(watu)) * rd(tu)
        x_tu = rd(wbau) * rd(au) + (1.0 - rd(wbtu)) * rd(tu)

        def mlp(x, w1, b1, w2, b2):
            h = jnp.maximum(
                jnp.dot(x, w1[...], preferred_element_type=jnp.float32)
                + b1[...], 0.0)
            return jnp.maximum(
                jnp.dot(h, w2[...], preferred_element_type=jnp.float32)
                + b2[...], 0.0)

        f_au = mlp(x_au, w1a, b1a, w2a, b2a)
        f_tu = mlp(x_tu, w1t, b1t, w2t, b2t)
        f_ai = mlp(rd(ai), w1i, b1i, w2i, b2i)
        f_ti = mlp(rd(ti), w1j, b1j, w2j, b2j)

        a_dot = jnp.sum(f_au * f_ai, axis=1, keepdims=True)
        t_dot = jnp.sum(f_tu * f_ti, axis=1, keepdims=True)
        a_s = jnp.maximum(a_dot, jnp.float32(1e-06))
        t_s = jnp.maximum(t_dot, jnp.float32(1e-06))
        pa = jnp.sum((a_s - ar[...]) ** 2) * inv_b
        pt = jnp.sum((t_s - tr[...]) ** 2) * inv_b

        i = pl.program_id(0)

        @pl.when(i == 0)
        def _():
            la[0, 0] = jnp.float32(0.0)
            lt[0, 0] = jnp.float32(0.0)

        la[0, 0] += pa
        lt[0, 0] += pt

    return body


def kernel(ausers, aitems, aratings, tusers, titems, tratings, params):
    B = ausers.shape[0]
    assert B % (NW * CHUNK) == 0
    n_chunks = (B // NW) // CHUNK

    tables = (params["a_emb_user"], params["t_emb_user"],
              params["a_emb_item"], params["t_emb_item"],
              params["W_a"], params["W_b"])

    packed = _sc_pack6()(*tables)

    iau, itu, iai, iti = (a.astype(jnp.int32)
                          for a in (ausers, tusers, aitems, titems))
    lines = [(a // 4).reshape(NW // 2, 8, CHUNK)
             for a in (iau, itu, iai, iti)]
    offs = [((a % 4) * D).reshape(NW // 2, 8, CHUNK)
            for a in (iau, itu, iai, iti)]

    g = _sc_gather8(B)(*packed, *lines, *offs)
    a_u, t_u, a_i, t_i, wa_au, wa_tu, wb_au, wb_tu = g

    NB = 8
    R = B // NB
    row = pl.BlockSpec((R // 8, 8, D), lambda i: (i, 0, 0))
    col = pl.BlockSpec((R, 1), lambda i: (i, 0))

    def full(shape):
        return pl.BlockSpec(shape, lambda i: tuple(0 for _ in shape))

    mlps = (params["mlp_a_users"], params["mlp_t_users"],
            params["mlp_a_items"], params["mlp_t_items"])
    wargs, wspecs = [], []
    for p in mlps:
        for arr in (p["W1"], p["b1"].reshape(1, -1),
                    p["W2"], p["b2"].reshape(1, -1)):
            wargs.append(arr)
            wspecs.append(full(arr.shape))

    ar2 = aratings.astype(jnp.float32).reshape(B, 1)
    tr2 = tratings.astype(jnp.float32).reshape(B, 1)

    la, lt = pl.pallas_call(
        _tc_combine(NB, 1.0 / B, R),
        grid=(NB,),
        in_specs=[row] * 8 + [col, col] + wspecs,
        out_specs=(pl.BlockSpec((1, 1), lambda i: (0, 0),
                                memory_space=pltpu.SMEM),) * 2,
        out_shape=(jax.ShapeDtypeStruct((1, 1), jnp.float32),) * 2,
    )(a_u, t_u, a_i, t_i, wa_au, wa_tu, wb_au, wb_tu, ar2, tr2, *wargs)

    return (la[0, 0], lt[0, 0])


# speedup vs baseline: 1.7638x; 1.6996x over previous
import functools

import jax
import jax.numpy as jnp
from jax import lax
from jax.experimental import pallas as pl
from jax.experimental.pallas import tpu as pltpu
from jax.experimental.pallas import tpu_sc as plsc

D = 32
NC = 2
NS = 16
NW = NC * NS
CHUNK = 128


def _sc_gathern(B, plan_ids):
    b_per_w = B // NW
    n_chunks = b_per_w // CHUNK
    n_out = len(plan_ids)
    mesh = plsc.VectorSubcoreMesh(core_axis_name="c", subcore_axis_name="s")
    out_type = [jax.ShapeDtypeStruct((B, D), jnp.float32)] * n_out
    scratch_types = [
        pltpu.VMEM((n_chunks, CHUNK), jnp.int32),
        pltpu.VMEM((n_chunks, CHUNK), jnp.int32),
        pltpu.VMEM((n_chunks, CHUNK), jnp.int32),
        pltpu.VMEM((b_per_w, D), jnp.float32),
        pltpu.VMEM((b_per_w, D), jnp.float32),
        pltpu.SemaphoreType.DMA,
        pltpu.SemaphoreType.DMA,
    ]

    @functools.partial(pl.kernel, mesh=mesh, out_type=out_type,
                       scratch_types=scratch_types,
                       compiler_params=pltpu.CompilerParams(
                           use_tc_tiling_on_sc=False))
    def k(*refs):
        tbls = refs[0:3]
        ih = refs[3:6]
        outs = refs[6:6 + n_out]
        iv = refs[6 + n_out:9 + n_out]
        buf0, buf1, sem0, sem1 = refs[9 + n_out:13 + n_out]
        wid = lax.axis_index("s") * NC + lax.axis_index("c")
        base = wid * b_per_w
        for h, v in zip(ih, iv):
            pltpu.sync_copy(h.at[wid], v)

        plan = [(tbls[ti], iv[ii]) for ti, ii in plan_ids]
        bufs = (buf0, buf1)
        sems = (sem0, sem1)

        def fire(g):
            tbl, idx = plan[g]
            buf, sem = bufs[g % 2], sems[g % 2]
            return [pltpu.async_copy(tbl.at[idx.at[c]],
                                     buf.at[pl.ds(c * CHUNK, CHUNK)], sem)
                    for c in range(n_chunks)]

        pending = fire(0)
        for g in range(n_out):
            for h in pending:
                h.wait()
            if g < n_out - 1:
                nxt = fire(g + 1)
            pltpu.sync_copy(bufs[g % 2], outs[g].at[pl.ds(base, b_per_w)])
            if g < n_out - 1:
                pending = nxt

    return k


def _tc_combine(nb):
    def body(au, tu, ai, ti, waau, watu, wbau, wbtu, ar, tr,
             w1a, b1a, w2a, b2a, w1t, b1t, w2t, b2t,
             w1i, b1i, w2i, b2i, w1j, b1j, w2j, b2j, la, lt):
        x_au = waau[...] * au[...] + (1.0 - watu[...]) * tu[...]
        x_tu = wbau[...] * au[...] + (1.0 - wbtu[...]) * tu[...]

        def mlp(x, w1, b1, w2, b2):
            h = jnp.maximum(
                jnp.dot(x, w1[...], preferred_element_type=jnp.float32)
                + b1[...], 0.0)
            return jnp.maximum(
                jnp.dot(h, w2[...], preferred_element_type=jnp.float32)
                + b2[...], 0.0)

        f_au = mlp(x_au, w1a, b1a, w2a, b2a)
        f_tu = mlp(x_tu, w1t, b1t, w2t, b2t)
        f_ai = mlp(ai[...], w1i, b1i, w2i, b2i)
        f_ti = mlp(ti[...], w1j, b1j, w2j, b2j)

        a_dot = jnp.sum(f_au * f_ai, axis=1, keepdims=True)
        t_dot = jnp.sum(f_tu * f_ti, axis=1, keepdims=True)
        a_s = jnp.maximum(a_dot, jnp.float32(1e-06))
        t_s = jnp.maximum(t_dot, jnp.float32(1e-06))
        pa = jnp.sum((a_s - ar[...]) ** 2)
        pt = jnp.sum((t_s - tr[...]) ** 2)

        i = pl.program_id(0)

        @pl.when(i == 0)
        def _():
            la[0, 0] = jnp.float32(0.0)
            lt[0, 0] = jnp.float32(0.0)

        la[0, 0] += pa
        lt[0, 0] += pt

    return body


def kernel(ausers, aitems, aratings, tusers, titems, tratings, params):
    B = ausers.shape[0]
    assert B % (NW * CHUNK) == 0
    n_chunks = (B // NW) // CHUNK

    au3, tu3, ai3, ti3 = (a.astype(jnp.int32).reshape(NW, n_chunks, CHUNK)
                          for a in (ausers, tusers, aitems, titems))

    a_u, wa_au, wa_tu, wb_au, wb_tu = _sc_gathern(
        B, ((0, 0), (1, 0), (1, 1), (2, 0), (2, 1)))(
        params["a_emb_user"], params["W_a"], params["W_b"],
        au3, tu3, tu3)
    t_u, a_i, t_i = _sc_gathern(
        B, ((0, 0), (1, 1), (2, 2)))(
        params["t_emb_user"], params["a_emb_item"], params["t_emb_item"],
        tu3, ai3, ti3)

    NB = 8
    R = B // NB
    row = pl.BlockSpec((R, D), lambda i: (i, 0))
    col = pl.BlockSpec((R, 1), lambda i: (i, 0))

    def full(shape):
        return pl.BlockSpec(shape, lambda i: tuple(0 for _ in shape))

    mlps = (params["mlp_a_users"], params["mlp_t_users"],
            params["mlp_a_items"], params["mlp_t_items"])
    wargs, wspecs = [], []
    for p in mlps:
        for nm, arr in (("W1", p["W1"]), ("b1", p["b1"].reshape(1, -1)),
                        ("W2", p["W2"]), ("b2", p["b2"].reshape(1, -1))):
            wargs.append(arr)
            wspecs.append(full(arr.shape))

    ar2 = aratings.astype(jnp.float32).reshape(B, 1)
    tr2 = tratings.astype(jnp.float32).reshape(B, 1)

    la, lt = pl.pallas_call(
        _tc_combine(NB),
        grid=(NB,),
        in_specs=[row] * 8 + [col, col] + wspecs,
        out_specs=(pl.BlockSpec((1, 1), lambda i: (0, 0),
                                memory_space=pltpu.SMEM),) * 2,
        out_shape=(jax.ShapeDtypeStruct((1, 1), jnp.float32),) * 2,
    )(a_u, t_u, a_i, t_i, wa_au, wa_tu, wb_au, wb_tu, ar2, tr2, *wargs)

    inv_b = jnp.float32(1.0 / B)
    return (la[0, 0] * inv_b, lt[0, 0] * inv_b)
